# baseline (speedup 1.0000x reference)
import functools

import jax
import jax.numpy as jnp
from jax.experimental import pallas as pl

EPS = 1e-09
K = 4


def _logmse_block(y_pred_ref, y_true_ref, out_ref):
    yp = y_pred_ref[...]
    yt = y_true_ref[...]
    lp = jnp.log2(jnp.maximum(yp, EPS))
    lt = jnp.log2(jnp.maximum(yt, EPS))
    d = lt - lp
    out_ref[...] = jnp.sum(d * d).reshape(1, 1)


def kernel(y_pred, y_true):
    b, s, _ = y_pred.shape
    lanes = 128
    reps = s // lanes
    n = b * reps
    yp = y_pred.reshape(n, lanes)
    yt = jnp.broadcast_to(y_true.reshape(b, 1, 1), (b, reps, 1)).reshape(n, 1)
    rows = n // K
    inv_n = 1.0 / float(b * s)
    partials = []
    for k in range(K):
        out = pl.pallas_call(
            _logmse_block,
            grid=(1,),
            in_specs=[
                pl.BlockSpec((rows, lanes), lambda i: (0, 0)),
                pl.BlockSpec((rows, 1), lambda i: (0, 0)),
            ],
            out_specs=pl.BlockSpec((1, 1), lambda i: (0, 0)),
            out_shape=jax.ShapeDtypeStruct((1, 1), jnp.float32),
        )(
            jax.lax.slice(yp, (k * rows, 0), ((k + 1) * rows, lanes)),
            jax.lax.slice(yt, (k * rows, 0), ((k + 1) * rows, 1)),
        )
        partials.append(out[0, 0])
    loss = (partials[0] + partials[1] + partials[2] + partials[3]) * inv_n
    return (loss, loss, jnp.array(0.0, dtype=jnp.float32))


# speedup vs baseline: 1.8540x; 1.8540x over previous
import functools

import jax
import jax.numpy as jnp
from jax.experimental import pallas as pl
from jax.experimental.pallas import tpu as pltpu

EPS = 1e-09


def _logmse_body(y_pred_ref, y_true_ref, out_ref, *, rows, nchunks, inv_n):
    def step(i, acc):
        yp = y_pred_ref[pl.ds(i * rows, rows), :]
        yt = y_true_ref[pl.ds(i * rows, rows), :]
        lp = jnp.log2(jnp.maximum(yp, EPS))
        lt = jnp.log2(jnp.maximum(yt, EPS))
        d = lt - lp
        return acc + jnp.sum(d * d)

    acc = jax.lax.fori_loop(0, nchunks, step, jnp.float32(0.0))
    out_ref[...] = (acc * inv_n).reshape(1, 1)


def kernel(y_pred, y_true):
    b, s, _ = y_pred.shape
    lanes = 128
    reps = s // lanes
    n = b * reps
    yp = y_pred.reshape(n, lanes)
    yt = jnp.broadcast_to(y_true.reshape(b, 1, 1), (b, reps, 1)).reshape(n, 1)
    rows = 4096
    nchunks = n // rows
    inv_n = 1.0 / float(b * s)
    out = pl.pallas_call(
        functools.partial(_logmse_body, rows=rows, nchunks=nchunks,
                          inv_n=inv_n),
        in_specs=[
            pl.BlockSpec(memory_space=pltpu.MemorySpace.VMEM),
            pl.BlockSpec(memory_space=pltpu.MemorySpace.VMEM),
        ],
        out_specs=pl.BlockSpec(memory_space=pltpu.MemorySpace.VMEM),
        out_shape=jax.ShapeDtypeStruct((1, 1), jnp.float32),
    )(yp, yt)
    loss = out[0, 0]
    return (loss, loss, jnp.array(0.0, dtype=jnp.float32))


# speedup vs baseline: 2.0439x; 1.1024x over previous
import functools

import jax
import jax.numpy as jnp
from jax.experimental import pallas as pl

EPS = 1e-09


def _logmse_block(y_pred_ref, y_true_ref, out_ref, *, nblocks, inv_n):
    i = pl.program_id(0)

    yp = y_pred_ref[...]
    yt = y_true_ref[...]
    lp = jnp.log2(jnp.maximum(yp, EPS))
    lt = jnp.log2(jnp.maximum(yt, EPS))
    d = lt - lp
    partial = jnp.sum(d * d).reshape(1, 1)

    @pl.when(i == 0)
    def _init():
        out_ref[...] = partial

    @pl.when(i > 0)
    def _acc():
        out_ref[...] = out_ref[...] + partial

    @pl.when(i == nblocks - 1)
    def _finish():
        out_ref[...] = out_ref[...] * inv_n


def kernel(y_pred, y_true):
    b, s, _ = y_pred.shape
    lanes = 128
    reps = s // lanes
    n = b * reps
    yp = y_pred.reshape(n, lanes)
    yt = jnp.broadcast_to(y_true.reshape(b, 1, 1), (b, reps, 1)).reshape(n, 1)
    rows = 16384
    nblocks = n // rows
    inv_n = 1.0 / float(b * s)
    out = pl.pallas_call(
        functools.partial(_logmse_block, nblocks=nblocks, inv_n=inv_n),
        grid=(nblocks,),
        in_specs=[
            pl.BlockSpec((rows, lanes), lambda i: (i, 0)),
            pl.BlockSpec((rows, 1), lambda i: (i, 0)),
        ],
        out_specs=pl.BlockSpec((1, 1), lambda i: (0, 0)),
        out_shape=jax.ShapeDtypeStruct((1, 1), jnp.float32),
    )(yp, yt)
    loss = out[0, 0]
    return (loss, loss, jnp.array(0.0, dtype=jnp.float32))


# speedup vs baseline: 2.0920x; 1.0235x over previous
import functools

import jax
import jax.numpy as jnp
from jax.experimental import pallas as pl

EPS = 1e-09


def _logmse_block(y_pred_ref, y_true_ref, out_ref, *, nblocks, inv_n):
    i = pl.program_id(0)

    yp = y_pred_ref[...]
    yt = y_true_ref[...]
    lp = jnp.log2(jnp.maximum(yp, EPS))
    lt = jnp.log2(jnp.maximum(yt, EPS))
    d = lt - lp
    partial = jnp.sum(d * d).reshape(1, 1)

    @pl.when(i == 0)
    def _init():
        out_ref[...] = partial

    @pl.when(i > 0)
    def _acc():
        out_ref[...] = out_ref[...] + partial

    @pl.when(i == nblocks - 1)
    def _finish():
        out_ref[...] = out_ref[...] * inv_n


def kernel(y_pred, y_true):
    b, s, _ = y_pred.shape
    lanes = 128
    reps = s // lanes
    n = b * reps
    yp = y_pred.reshape(n, lanes)
    yt = jnp.broadcast_to(y_true.reshape(b, 1, 1), (b, reps, 1)).reshape(n, 1)
    rows = 8192
    nblocks = n // rows
    inv_n = 1.0 / float(b * s)
    out = pl.pallas_call(
        functools.partial(_logmse_block, nblocks=nblocks, inv_n=inv_n),
        grid=(nblocks,),
        in_specs=[
            pl.BlockSpec((rows, lanes), lambda i: (i, 0)),
            pl.BlockSpec((rows, 1), lambda i: (i, 0)),
        ],
        out_specs=pl.BlockSpec((1, 1), lambda i: (0, 0)),
        out_shape=jax.ShapeDtypeStruct((1, 1), jnp.float32),
    )(yp, yt)
    loss = out[0, 0]
    return (loss, loss, jnp.array(0.0, dtype=jnp.float32))
